# baseline (speedup 1.0000x reference)
import math

import jax
import jax.numpy as jnp
from jax import lax
from jax.experimental import pallas as pl
from jax.experimental.pallas import tpu as pltpu

HID, NH, HD = 768, 12, 64
NB, NP = 8, 196
KS = max(1, int(NP * 0.7))
MW = NP - KS
_HI = lax.Precision.HIGHEST
_DE = lax.Precision.DEFAULT


def _dot_nt(a, b, prec):
    return lax.dot_general(a, b, (((1,), (1,)), ((), ())), precision=prec,
                           preferred_element_type=jnp.float32)


def _dot_nn(a, b, prec):
    return lax.dot_general(a, b, (((1,), (0,)), ((), ())), precision=prec,
                           preferred_element_type=jnp.float32)


def _grid_kernel(q_ref, img_ref, wq_ref, wk_ref, imgb_ref, o_ref, s_scr):
    b = pl.program_id(0)

    @pl.when(b == 0)
    def _phase_a():
        rows = {}
        for bb in range(NB):
            hs = [h for h in range(NH) if h % NB == bb or (h + 4) % NB == bb]
            wk_cat = jnp.concatenate([wk_ref[h] for h in hs], axis=1)
            wq_cat = jnp.concatenate([wq_ref[h] for h in hs], axis=1)
            kx_cat = _dot_nn(img_ref[bb], wk_cat, _DE)
            qx_cat = _dot_nn(q_ref[bb:bb + 1, :], wq_cat, _DE)
            for i, h in enumerate(hs):
                kx = kx_cat[:, i * HD:(i + 1) * HD]
                qx = qx_cat[:, i * HD:(i + 1) * HD]
                logit = _dot_nt(qx, kx, _DE) / math.sqrt(HD)
                mx = jnp.max(logit, axis=1, keepdims=True)
                e = jnp.exp(logit - mx)
                rows[(bb, h)] = e / jnp.sum(e, axis=1, keepdims=True)
        for par in (0, 1):
            acc = rows[((4 * par) % NB, 0)]
            for h in range(1, NH):
                acc = acc + rows[((4 * par + h) % NB, h)]
            s_scr[par:par + 1, :] = acc / float(NH)

    par = lax.rem(b, 2)
    s_row = s_scr[pl.ds(par, 1), :]

    lt_mn = (lax.broadcasted_iota(jnp.int32, (NP, NP), 0)
             < lax.broadcasted_iota(jnp.int32, (NP, NP), 1))
    ident = (lax.broadcasted_iota(jnp.int32, (NP, NP), 0)
             == lax.broadcasted_iota(jnp.int32, (NP, NP), 1)).astype(jnp.float32)
    krow_s = lax.broadcasted_iota(jnp.int32, (KS, NP), 0).astype(jnp.float32)
    krow_w = (lax.broadcasted_iota(jnp.int32, (MW, NP), 0).astype(jnp.float32)
              + float(KS))
    s_col = _dot_nt(ident, s_row, _HI)
    sc = jnp.broadcast_to(s_col, (NP, NP))
    sr = jnp.broadcast_to(s_row, (NP, NP))
    cmp = (sc > sr) | ((sc == sr) & lt_mn)
    rank_row = jnp.sum(cmp.astype(jnp.float32), axis=0, keepdims=True)
    onehotS = (krow_s == jnp.broadcast_to(rank_row, (KS, NP))).astype(jnp.float32)
    onehotW = (krow_w == jnp.broadcast_to(rank_row, (MW, NP))).astype(jnp.float32)

    slot_iota_w = lax.broadcasted_iota(jnp.int32, (MW, KS), 1).astype(jnp.float32)
    lt_w = (lax.broadcasted_iota(jnp.int32, (MW, MW), 0)
            < lax.broadcasted_iota(jnp.int32, (MW, MW), 1))
    riota_w = lax.broadcasted_iota(jnp.int32, (MW, MW), 0).astype(jnp.float32)
    kcol = lax.broadcasted_iota(jnp.int32, (KS, 1), 0).astype(jnp.float32)
    ones11 = jnp.ones((1, 1), jnp.float32)

    img = imgb_ref[0]
    v_s = _dot_nn(onehotS, img, _HI)
    v_w = _dot_nn(onehotW, img, _HI)
    vsn = v_s / jnp.maximum(
        jnp.sqrt(jnp.sum(v_s * v_s, axis=1, keepdims=True)), 1e-12)
    vwn = v_w / jnp.maximum(
        jnp.sqrt(jnp.sum(v_w * v_w, axis=1, keepdims=True)), 1e-12)
    sim = _dot_nt(vwn, vsn, _DE)
    best = jnp.max(sim, axis=1, keepdims=True)
    pos = jnp.where(sim == best, slot_iota_w, 1e9)
    assign = jnp.min(pos, axis=1, keepdims=True)
    es = jnp.exp(best)
    theta_w = es / (es + math.e)

    a_row = _dot_nt(ones11, assign, _HI)
    a_colb = jnp.broadcast_to(assign, (MW, MW))
    a_rowb = jnp.broadcast_to(a_row, (MW, MW))
    scmp = (a_colb < a_rowb) | ((a_colb == a_rowb) & lt_w)
    srank = jnp.sum(scmp.astype(jnp.float32), axis=0, keepdims=True)
    P = (riota_w == jnp.broadcast_to(srank, (MW, MW))).astype(jnp.float32)
    sv = _dot_nn(P, v_w, _HI)
    sk = _dot_nn(P, assign, _HI)
    st = _dot_nn(P, theta_w, _HI)

    t = 1
    while t < MW:
        padv = jnp.full((t, HID), -1e4, jnp.float32)
        padk = jnp.full((t, 1), -1.0, jnp.float32)
        shv = jnp.concatenate([padv, sv[:MW - t]], axis=0)
        shk = jnp.concatenate([padk, sk[:MW - t]], axis=0)
        sht = jnp.concatenate([padk, st[:MW - t]], axis=0)
        same = shk == sk
        sv = jnp.where(same, jnp.maximum(sv, shv), sv)
        st = jnp.where(same, jnp.maximum(st, sht), st)
        t *= 2

    nk = jnp.concatenate([sk[1:], jnp.full((1, 1), -1.0, jnp.float32)],
                         axis=0)
    run_end = sk != nk
    sk_row = _dot_nt(ones11, sk, _HI)
    re_row = _dot_nt(ones11, run_end.astype(jnp.float32), _HI)
    M = ((kcol == jnp.broadcast_to(sk_row, (KS, MW)))
         & (jnp.broadcast_to(re_row, (KS, MW)) > 0.0)).astype(jnp.float32)
    attended = _dot_nn(M, sv, _HI)
    theta_s = _dot_nn(M, st, _HI)
    o_ref[0] = (1.0 - theta_s) * v_s + theta_s * attended


def kernel(text_hidden_states, image_hidden_states, w_kx, w_qx, proj_w, proj_b):
    del proj_w, proj_b
    q = text_hidden_states[:, 0, :]
    return pl.pallas_call(
        _grid_kernel,
        grid=(NB,),
        in_specs=[
            pl.BlockSpec((NB, HID), lambda b: (0, 0)),
            pl.BlockSpec((NB, NP, HID), lambda b: (0, 0, 0)),
            pl.BlockSpec((NH, HID, HD), lambda b: (0, 0, 0)),
            pl.BlockSpec((NH, HID, HD), lambda b: (0, 0, 0)),
            pl.BlockSpec((1, NP, HID), lambda b: (b, 0, 0)),
        ],
        out_specs=pl.BlockSpec((1, KS, HID), lambda b: (b, 0, 0)),
        out_shape=jax.ShapeDtypeStruct((NB, KS, HID), jnp.float32),
        scratch_shapes=[pltpu.VMEM((2, NP), jnp.float32)],
    )(q, image_hidden_states, w_qx, w_kx, image_hidden_states)


# speedup vs baseline: 1.0334x; 1.0334x over previous
import math

import jax
import jax.numpy as jnp
from jax import lax
from jax.experimental import pallas as pl

HID, NH, HD = 768, 12, 64
NB, NP = 8, 196
KS = max(1, int(NP * 0.7))
MW = NP - KS
_HI = lax.Precision.HIGHEST
_DE = lax.Precision.DEFAULT


def _dot_nt(a, b, prec):
    return lax.dot_general(a, b, (((1,), (1,)), ((), ())), precision=prec,
                           preferred_element_type=jnp.float32)


def _dot_nn(a, b, prec):
    return lax.dot_general(a, b, (((1,), (0,)), ((), ())), precision=prec,
                           preferred_element_type=jnp.float32)


def _full_kernel(q_ref, img_ref, wq_ref, wk_ref, o_ref):
    rows = {}
    for bb in range(NB):
        hs = [h for h in range(NH) if h % NB == bb or (h + 4) % NB == bb]
        wk_cat = jnp.concatenate([wk_ref[h] for h in hs], axis=1)
        wq_cat = jnp.concatenate([wq_ref[h] for h in hs], axis=1)
        kx_cat = _dot_nn(img_ref[bb], wk_cat, _DE)
        qx_cat = _dot_nn(q_ref[bb:bb + 1, :], wq_cat, _DE)
        for i, h in enumerate(hs):
            kx = kx_cat[:, i * HD:(i + 1) * HD]
            qx = qx_cat[:, i * HD:(i + 1) * HD]
            logit = _dot_nt(qx, kx, _DE) / math.sqrt(HD)
            mx = jnp.max(logit, axis=1, keepdims=True)
            e = jnp.exp(logit - mx)
            rows[(bb, h)] = e / jnp.sum(e, axis=1, keepdims=True)
    s_par = []
    for par in (0, 1):
        acc = rows[((4 * par) % NB, 0)]
        for h in range(1, NH):
            acc = acc + rows[((4 * par + h) % NB, h)]
        s_par.append(acc / float(NH))

    lt_mn = (lax.broadcasted_iota(jnp.int32, (NP, NP), 0)
             < lax.broadcasted_iota(jnp.int32, (NP, NP), 1))
    ident = (lax.broadcasted_iota(jnp.int32, (NP, NP), 0)
             == lax.broadcasted_iota(jnp.int32, (NP, NP), 1)).astype(jnp.float32)
    krow_s = lax.broadcasted_iota(jnp.int32, (KS, NP), 0).astype(jnp.float32)
    krow_w = (lax.broadcasted_iota(jnp.int32, (MW, NP), 0).astype(jnp.float32)
              + float(KS))
    onehots = []
    for par in (0, 1):
        s_row = s_par[par]
        s_col = _dot_nt(ident, s_row, _HI)
        sc = jnp.broadcast_to(s_col, (NP, NP))
        sr = jnp.broadcast_to(s_row, (NP, NP))
        cmp = (sc > sr) | ((sc == sr) & lt_mn)
        rank_row = jnp.sum(cmp.astype(jnp.float32), axis=0, keepdims=True)
        rank_b = jnp.broadcast_to(rank_row, (KS, NP))
        onehotS = (krow_s == rank_b).astype(jnp.float32)
        rank_bw = jnp.broadcast_to(rank_row, (MW, NP))
        onehotW = (krow_w == rank_bw).astype(jnp.float32)
        onehots.append((onehotS, onehotW))

    slot_iota_w = lax.broadcasted_iota(jnp.int32, (MW, KS), 1).astype(jnp.float32)
    lt_w = (lax.broadcasted_iota(jnp.int32, (MW, MW), 0)
            < lax.broadcasted_iota(jnp.int32, (MW, MW), 1))
    riota_w = lax.broadcasted_iota(jnp.int32, (MW, MW), 0).astype(jnp.float32)
    kcol = lax.broadcasted_iota(jnp.int32, (KS, 1), 0).astype(jnp.float32)
    ones11 = jnp.ones((1, 1), jnp.float32)
    for b in range(NB):
        onehotS, onehotW = onehots[b % 2]
        img = img_ref[b]
        v_s = _dot_nn(onehotS, img, _HI)
        v_w = _dot_nn(onehotW, img, _HI)
        vsn = v_s / jnp.maximum(
            jnp.sqrt(jnp.sum(v_s * v_s, axis=1, keepdims=True)), 1e-12)
        vwn = v_w / jnp.maximum(
            jnp.sqrt(jnp.sum(v_w * v_w, axis=1, keepdims=True)), 1e-12)
        sim = _dot_nt(vwn, vsn, _DE)
        best = jnp.max(sim, axis=1, keepdims=True)
        pos = jnp.where(sim == best, slot_iota_w, 1e9)
        assign = jnp.min(pos, axis=1, keepdims=True)
        es = jnp.exp(best)
        theta_w = es / (es + math.e)

        a_row = _dot_nt(ones11, assign, _HI)
        a_colb = jnp.broadcast_to(assign, (MW, MW))
        a_rowb = jnp.broadcast_to(a_row, (MW, MW))
        scmp = (a_colb < a_rowb) | ((a_colb == a_rowb) & lt_w)
        srank = jnp.sum(scmp.astype(jnp.float32), axis=0, keepdims=True)
        P = (riota_w == jnp.broadcast_to(srank, (MW, MW))).astype(jnp.float32)
        sv = _dot_nn(P, v_w, _HI)
        sk = _dot_nn(P, assign, _HI)
        st = _dot_nn(P, theta_w, _HI)

        t = 1
        while t < MW:
            padv = jnp.full((t, HID), -1e4, jnp.float32)
            padk = jnp.full((t, 1), -1.0, jnp.float32)
            shv = jnp.concatenate([padv, sv[:MW - t]], axis=0)
            shk = jnp.concatenate([padk, sk[:MW - t]], axis=0)
            sht = jnp.concatenate([padk, st[:MW - t]], axis=0)
            same = shk == sk
            sv = jnp.where(same, jnp.maximum(sv, shv), sv)
            st = jnp.where(same, jnp.maximum(st, sht), st)
            t *= 2

        nk = jnp.concatenate([sk[1:], jnp.full((1, 1), -1.0, jnp.float32)],
                             axis=0)
        run_end = sk != nk
        sk_row = _dot_nt(ones11, sk, _HI)
        re_row = _dot_nt(ones11, run_end.astype(jnp.float32), _HI)
        M = ((kcol == jnp.broadcast_to(sk_row, (KS, MW)))
             & (jnp.broadcast_to(re_row, (KS, MW)) > 0.0)).astype(jnp.float32)
        attended = _dot_nn(M, sv, _HI)
        theta_s = _dot_nn(M, st, _HI)
        o_ref[b] = (1.0 - theta_s) * v_s + theta_s * attended


def kernel(text_hidden_states, image_hidden_states, w_kx, w_qx, proj_w, proj_b):
    del proj_w, proj_b
    q = text_hidden_states[:, 0, :]
    return pl.pallas_call(
        _full_kernel,
        out_shape=jax.ShapeDtypeStruct((NB, KS, HID), jnp.float32),
    )(q, image_hidden_states, w_qx, w_kx)
